# baseline (speedup 1.0000x reference)
import jax
import jax.numpy as jnp
from jax.experimental import pallas as pl
from jax.experimental.pallas import tpu as pltpu

_TOKENS = 16384
_D = 2048
_E = 64
_BT = 2048


def _router_body(x_ref, wt_ref, probs_ref, gates_ref, idx_ref):
    x = x_ref[...]
    wt = wt_ref[...]
    logits = jax.lax.dot_general(
        x, wt, (((1,), (0,)), ((), ())),
        preferred_element_type=jnp.float32,
        precision=jax.lax.Precision.DEFAULT,
    )
    m = jnp.max(logits, axis=-1, keepdims=True)
    e = jnp.exp(logits - m)
    probs = e / jnp.sum(e, axis=-1, keepdims=True)
    probs_ref[...] = probs

    lane = jax.lax.broadcasted_iota(jnp.int32, probs.shape, 1)
    i1 = jnp.argmax(probs, axis=-1, keepdims=True)
    m1 = jnp.max(probs, axis=-1, keepdims=True)
    masked = jnp.where(lane == i1, -1.0, probs)
    i2 = jnp.argmax(masked, axis=-1, keepdims=True)
    m2 = jnp.max(masked, axis=-1, keepdims=True)
    s = m1 + m2
    gates_ref[...] = jnp.concatenate([m1 / s, m2 / s], axis=1)
    idx_ref[...] = jnp.concatenate([i1, i2], axis=1)


def kernel(x, W_r):
    wt = W_r.T
    grid = (_TOKENS // _BT,)
    probs, gates, idx = pl.pallas_call(
        _router_body,
        grid=grid,
        compiler_params=pltpu.CompilerParams(
            dimension_semantics=("parallel",),
        ),
        in_specs=[
            pl.BlockSpec((_BT, _D), lambda i: (i, 0)),
            pl.BlockSpec((_D, _E), lambda i: (0, 0)),
        ],
        out_specs=[
            pl.BlockSpec((_BT, _E), lambda i: (i, 0)),
            pl.BlockSpec((_BT, 2), lambda i: (i, 0)),
            pl.BlockSpec((_BT, 2), lambda i: (i, 0)),
        ],
        out_shape=[
            jax.ShapeDtypeStruct((_TOKENS, _E), jnp.float32),
            jax.ShapeDtypeStruct((_TOKENS, 2), jnp.float32),
            jax.ShapeDtypeStruct((_TOKENS, 2), jnp.int32),
        ],
    )(x, wt)
    return (gates, idx, probs)


# speedup vs baseline: 1.0018x; 1.0018x over previous
import jax
import jax.numpy as jnp
from jax.experimental import pallas as pl

_TOKENS = 16384
_D = 2048
_E = 64
_BT = 2048


def _router_body(x_ref, wt_ref, probs_ref, gates_ref, idx_ref):
    x = x_ref[...]
    wt = wt_ref[...]
    logits = jax.lax.dot_general(
        x, wt, (((1,), (0,)), ((), ())),
        preferred_element_type=jnp.float32,
        precision=jax.lax.Precision.DEFAULT,
    )
    lane = jax.lax.broadcasted_iota(jnp.int32, logits.shape, 1)
    i1 = jnp.argmax(logits, axis=-1, keepdims=True)
    m = jnp.max(logits, axis=-1, keepdims=True)
    neg = jnp.finfo(jnp.float32).min
    masked = jnp.where(lane == i1, neg, logits)
    i2 = jnp.argmax(masked, axis=-1, keepdims=True)
    m2 = jnp.max(masked, axis=-1, keepdims=True)

    e = jnp.exp(logits - m)
    probs_ref[...] = e / jnp.sum(e, axis=-1, keepdims=True)

    e2 = jnp.exp(m2 - m)
    g1 = 1.0 / (1.0 + e2)
    gates_ref[...] = jnp.concatenate([g1, e2 * g1], axis=1)
    idx_ref[...] = jnp.concatenate([i1, i2], axis=1)


def kernel(x, W_r):
    wt = W_r.T
    grid = (_TOKENS // _BT,)
    probs, gates, idx = pl.pallas_call(
        _router_body,
        grid=grid,
        in_specs=[
            pl.BlockSpec((_BT, _D), lambda i: (i, 0)),
            pl.BlockSpec((_D, _E), lambda i: (0, 0)),
        ],
        out_specs=[
            pl.BlockSpec((_BT, _E), lambda i: (i, 0)),
            pl.BlockSpec((_BT, 2), lambda i: (i, 0)),
            pl.BlockSpec((_BT, 2), lambda i: (i, 0)),
        ],
        out_shape=[
            jax.ShapeDtypeStruct((_TOKENS, _E), jnp.float32),
            jax.ShapeDtypeStruct((_TOKENS, 2), jnp.float32),
            jax.ShapeDtypeStruct((_TOKENS, 2), jnp.int32),
        ],
    )(x, wt)
    return (gates, idx, probs)


# speedup vs baseline: 1.0080x; 1.0062x over previous
import jax
import jax.numpy as jnp
from jax.experimental import pallas as pl

_TOKENS = 16384
_D = 2048
_E = 64
_BT = 2048


_SUB = 256


def _router_body(x_ref, wt_ref, probs_ref, gates_ref, idx_ref):
    wt = wt_ref[...]
    for j in range(_BT // _SUB):
        sl = pl.ds(j * _SUB, _SUB)
        logits = jax.lax.dot_general(
            x_ref[sl, :], wt, (((1,), (0,)), ((), ())),
            preferred_element_type=jnp.float32,
            precision=jax.lax.Precision.DEFAULT,
        )
        lane = jax.lax.broadcasted_iota(jnp.int32, logits.shape, 1)
        i1 = jnp.argmax(logits, axis=-1, keepdims=True)
        m = jnp.max(logits, axis=-1, keepdims=True)
        neg = jnp.finfo(jnp.float32).min
        masked = jnp.where(lane == i1, neg, logits)
        i2 = jnp.argmax(masked, axis=-1, keepdims=True)
        m2 = jnp.max(masked, axis=-1, keepdims=True)

        e = jnp.exp(logits - m)
        probs_ref[sl, :] = e / jnp.sum(e, axis=-1, keepdims=True)

        e2 = jnp.exp(m2 - m)
        g1 = 1.0 / (1.0 + e2)
        gates_ref[sl, :] = jnp.concatenate([g1, e2 * g1], axis=1)
        idx_ref[sl, :] = jnp.concatenate([i1, i2], axis=1)


def kernel(x, W_r):
    wt = W_r.T
    grid = (_TOKENS // _BT,)
    probs, gates, idx = pl.pallas_call(
        _router_body,
        grid=grid,
        in_specs=[
            pl.BlockSpec((_BT, _D), lambda i: (i, 0)),
            pl.BlockSpec((_D, _E), lambda i: (0, 0)),
        ],
        out_specs=[
            pl.BlockSpec((_BT, _E), lambda i: (i, 0)),
            pl.BlockSpec((_BT, 2), lambda i: (i, 0)),
            pl.BlockSpec((_BT, 2), lambda i: (i, 0)),
        ],
        out_shape=[
            jax.ShapeDtypeStruct((_TOKENS, _E), jnp.float32),
            jax.ShapeDtypeStruct((_TOKENS, 2), jnp.float32),
            jax.ShapeDtypeStruct((_TOKENS, 2), jnp.int32),
        ],
    )(x, wt)
    return (gates, idx, probs)
